# baseline (speedup 1.0000x reference)
import functools

import jax
import jax.numpy as jnp
from jax import lax
from jax.experimental import pallas as pl
from jax.experimental.pallas import tpu as pltpu
from jax.experimental.pallas import tpu_sc as plsc

N_NODES = 10000
N_PAD = 10240
E_EDGES = 640000
NW = 32
CHUNK = 128
CH_PER_W = 158
EPW = CHUNK * CH_PER_W
E_PAD = EPW * NW
TW = 128
TE = 2048
TN = 1024
DUMMY = N_NODES

_MESH = dict(core_axis_name="c", subcore_axis_name="s", num_cores=2,
             num_subcores=16)


def _silu(x):
    return x * (1.0 / (1.0 + jnp.exp(-x)))


def _gather_body(ch, table, idx_s, idx_e, xs_out, xe_out,
                 idxs_v, idxe_v, bufs_a, bufs_b, sems_a, sems_b):
    cid = lax.axis_index("c")
    sid = lax.axis_index("s")
    wid = cid * 16 + sid
    pltpu.sync_copy(idx_s.at[wid], idxs_v)
    pltpu.sync_copy(idx_e.at[wid], idxe_v)
    base0 = wid * (ch * CHUNK)

    def fire(j, slot):
        pltpu.async_copy(table.at[idxs_v.at[j]], bufs_a[slot], sems_a[slot])
        pltpu.async_copy(table.at[idxe_v.at[j]], bufs_b[slot], sems_b[slot])

    def drain(j, slot):
        pltpu.make_async_copy(table.at[idxs_v.at[j]], bufs_a[slot],
                              sems_a[slot]).wait()
        pltpu.make_async_copy(table.at[idxe_v.at[j]], bufs_b[slot],
                              sems_b[slot]).wait()
        base = base0 + j * CHUNK
        pltpu.sync_copy(bufs_a[slot], xs_out.at[pl.ds(base, CHUNK)])
        pltpu.sync_copy(bufs_b[slot], xe_out.at[pl.ds(base, CHUNK)])

    fire(0, 0)
    fire(1, 1)

    def body(jj, carry):
        for b in range(2):
            j = jj * 2 + b
            drain(j, b)

            @pl.when(jj < ch // 2 - 1)
            def _():
                fire(j + 2, b)
        return carry

    lax.fori_loop(0, ch // 2, body, 0)


@functools.lru_cache(maxsize=None)
def _make_gather(ch):
    ne = NW * ch * CHUNK
    return functools.partial(
        pl.kernel,
        out_type=[jax.ShapeDtypeStruct((ne, TW), jnp.float32),
                  jax.ShapeDtypeStruct((ne, TW), jnp.float32)],
        mesh=plsc.VectorSubcoreMesh(**_MESH),
        scratch_types=[pltpu.VMEM((ch, CHUNK), jnp.int32),
                       pltpu.VMEM((ch, CHUNK), jnp.int32),
                       [pltpu.VMEM((CHUNK, TW), jnp.float32)] * 2,
                       [pltpu.VMEM((CHUNK, TW), jnp.float32)] * 2,
                       [pltpu.SemaphoreType.DMA] * 2,
                       [pltpu.SemaphoreType.DMA] * 2],
    )(functools.partial(_gather_body, ch))


CHUNK_SC = 128
CH_SC = EPW // CHUNK_SC


def _scatter_body(chsc, mc_in, idx_s, z128, p_out, acc, idx_bufs,
                  bufs, sems, isems):
    cid = lax.axis_index("c")
    sid = lax.axis_index("s")
    wid = cid * 16 + sid
    r0 = sid * (N_PAD // 16)
    nr = N_PAD // 16
    pltpu.sync_copy(z128.at[pl.ds(r0, nr)], acc.at[pl.ds(r0, nr)])
    plsc.subcore_barrier()
    base0 = wid * (chsc * CHUNK_SC)

    def fire(j, slot):
        base = base0 + j * CHUNK_SC
        pltpu.async_copy(mc_in.at[pl.ds(base, CHUNK_SC)], bufs[slot], sems[slot])
        pltpu.async_copy(idx_s.at[wid, j], idx_bufs[slot], isems[slot])

    def drain(j, slot):
        base = base0 + j * CHUNK_SC
        pltpu.make_async_copy(mc_in.at[pl.ds(base, CHUNK_SC)], bufs[slot],
                              sems[slot]).wait()
        pltpu.make_async_copy(idx_s.at[wid, j], idx_bufs[slot],
                              isems[slot]).wait()
        pltpu.sync_copy(bufs[slot], acc.at[idx_bufs[slot]], add=True)

    fire(0, 0)
    fire(1, 1)

    def body(jj, carry):
        for b in range(2):
            j = jj * 2 + b
            drain(j, b)

            @pl.when(jj < chsc // 2 - 1)
            def _():
                fire(j + 2, b)
        return carry

    lax.fori_loop(0, chsc // 2, body, 0)
    plsc.subcore_barrier()
    pltpu.sync_copy(acc.at[pl.ds(r0, nr)], p_out.at[cid, pl.ds(r0, nr)])


@functools.lru_cache(maxsize=None)
def _make_scatter(chsc):
    return functools.partial(
        pl.kernel,
        out_type=jax.ShapeDtypeStruct((2, N_PAD, 128), jnp.float32),
        mesh=plsc.VectorSubcoreMesh(**_MESH),
        scratch_types=[pltpu.VMEM_SHARED((N_PAD, 128), jnp.float32),
                       [pltpu.VMEM((CHUNK_SC,), jnp.int32)] * 2,
                       [pltpu.VMEM((CHUNK_SC, 128), jnp.float32)] * 2,
                       [pltpu.SemaphoreType.DMA] * 2,
                       [pltpu.SemaphoreType.DMA] * 2],
    )(functools.partial(_scatter_body, chsc))


def _init_body(feats, cexp, wemb, bemb, out):
    h0 = jnp.dot(feats[...], wemb[...],
                 preferred_element_type=jnp.float32) + bemb[...]
    out[...] = jnp.concatenate(
        [h0, cexp[:, :3], jnp.zeros((h0.shape[0], 61), jnp.float32)], axis=1)


def _init_table(feats_pad, cexp, wemb, bemb):
    return pl.pallas_call(
        _init_body,
        grid=(N_PAD // TN,),
        in_specs=[pl.BlockSpec((TN, 128), lambda i: (i, 0)),
                  pl.BlockSpec((TN, 16), lambda i: (i, 0)),
                  pl.BlockSpec((128, 64), lambda i: (0, 0)),
                  pl.BlockSpec((1, 64), lambda i: (0, 0))],
        out_specs=pl.BlockSpec((TN, TW), lambda i: (i, 0)),
        out_shape=jax.ShapeDtypeStruct((N_PAD, TW), jnp.float32),
    )(feats_pad, cexp, wemb, bemb)


def _edge_body(xs, xe, eft, w1h, w1e, w1n, w1f, b1, w2, b2, c1, c1b, c2p, c2bp,
               mc_out):
    hs = xs[:, :64]
    he = xe[:, :64]
    cd = xs[:, 64:67] - xe[:, 64:67]
    norm = jnp.sqrt(jnp.sum(cd * cd, axis=1, keepdims=True))
    mp = (jnp.dot(hs, w1h[...], preferred_element_type=jnp.float32)
          + jnp.dot(he, w1e[...], preferred_element_type=jnp.float32)
          + lax.dot_general(eft[...], w1f[...], (((0,), (0,)), ((), ())),
                            preferred_element_type=jnp.float32)
          + norm * w1n[...] + b1[...])
    m = _silu(mp)
    m = _silu(jnp.dot(m, w2[...], preferred_element_type=jnp.float32) + b2[...])
    ch = _silu(jnp.dot(m, c1[...], preferred_element_type=jnp.float32) + c1b[...])
    cm = jnp.dot(ch, c2p[...], preferred_element_type=jnp.float32) + c2bp[...]
    cdcm = cd * cm[:, :3]
    nrow = cd.shape[0]
    mc_out[...] = jnp.concatenate(
        [m, cdcm, jnp.ones((nrow, 1), jnp.float32),
         jnp.zeros((nrow, 60), jnp.float32)], axis=1)


def _edge_mlp(xs, xe, eft, w):
    ne = xs.shape[0]
    wspec = lambda shp: pl.BlockSpec(shp, lambda i: (0, 0))
    return pl.pallas_call(
        _edge_body,
        grid=(ne // TE,),
        in_specs=[pl.BlockSpec((TE, TW), lambda i: (i, 0)),
                  pl.BlockSpec((TE, TW), lambda i: (i, 0)),
                  pl.BlockSpec((8, TE), lambda i: (0, i)),
                  wspec((64, 64)), wspec((64, 64)), wspec((1, 64)),
                  wspec((8, 64)), wspec((1, 64)), wspec((64, 64)),
                  wspec((1, 64)), wspec((64, 64)), wspec((1, 64)),
                  wspec((64, 8)), wspec((1, 8))],
        out_specs=pl.BlockSpec((TE, 128), lambda i: (i, 0)),
        out_shape=jax.ShapeDtypeStruct((ne, 128), jnp.float32),
    )(xs, xe, eft, w['w1h'], w['w1e'], w['w1n'], w['w1f'], w['b1'],
      w['w2'], w['b2'], w['c1'], w['c1b'], w['c2p'], w['c2bp'])


def _node_body(tab, vels, pm0, pm1, v1, v1b, v2p, v2bp,
               n1h, n1a, n1b, n2, n2b, out):
    h = tab[:, :64]
    coords = tab[:, 64:67]
    agg = pm0[...] + pm1[...]
    aggm = agg[:, :64]
    aggc = agg[:, 64:67]
    cnt = jnp.maximum(agg[:, 67:68], 1.0)
    vs = _silu(jnp.dot(h, v1[...], preferred_element_type=jnp.float32) + v1b[...])
    vs = jnp.dot(vs, v2p[...], preferred_element_type=jnp.float32) + v2bp[...]
    newc = coords + aggc[:, :3] / cnt + vs[:, :1] * vels[:, :3]
    u = _silu(jnp.dot(h, n1h[...], preferred_element_type=jnp.float32)
              + jnp.dot(aggm, n1a[...], preferred_element_type=jnp.float32)
              + n1b[...])
    u = jnp.dot(u, n2[...], preferred_element_type=jnp.float32) + n2b[...]
    nrow = h.shape[0]
    out[...] = jnp.concatenate(
        [h + u, newc, jnp.zeros((nrow, 61), jnp.float32)], axis=1)


def _node_update(tab, vels_pad, pa, w):
    wspec = lambda shp: pl.BlockSpec(shp, lambda i: (0, 0))
    return pl.pallas_call(
        _node_body,
        grid=(N_PAD // TN,),
        in_specs=[pl.BlockSpec((TN, TW), lambda i: (i, 0)),
                  pl.BlockSpec((TN, 8), lambda i: (i, 0)),
                  pl.BlockSpec((TN, 128), lambda i: (i, 0)),
                  pl.BlockSpec((TN, 128), lambda i: (i, 0)),
                  wspec((64, 64)), wspec((1, 64)), wspec((64, 8)),
                  wspec((1, 8)), wspec((64, 64)), wspec((64, 64)),
                  wspec((1, 64)), wspec((64, 64)), wspec((1, 64))],
        out_specs=pl.BlockSpec((TN, TW), lambda i: (i, 0)),
        out_shape=jax.ShapeDtypeStruct((N_PAD, TW), jnp.float32),
    )(tab, vels_pad, pa[0], pa[1],
      w['v1'], w['v1b'], w['v2p'], w['v2bp'],
      w['n1h'], w['n1a'], w['n1b'], w['n2'], w['n2b'])


def _prep_block_weights(p):
    w1, b1 = p['e1']
    f32 = jnp.float32
    w = {
        'w1h': w1[:64],
        'w1e': w1[64:128],
        'w1n': w1[128:129],
        'w1f': jnp.zeros((8, 64), f32).at[:4].set(w1[129:133]),
        'b1': b1[None, :],
        'w2': p['e2'][0], 'b2': p['e2'][1][None, :],
        'c1': p['c1'][0], 'c1b': p['c1'][1][None, :],
        'c2p': jnp.zeros((64, 8), f32).at[:, :3].set(p['c2'][0]),
        'c2bp': jnp.zeros((1, 8), f32).at[0, :3].set(p['c2'][1]),
        'v1': p['v1'][0], 'v1b': p['v1'][1][None, :],
        'v2p': jnp.zeros((64, 8), f32).at[:, :1].set(p['v2'][0]),
        'v2bp': jnp.zeros((1, 8), f32).at[0, :1].set(p['v2'][1]),
        'n1h': p['n1'][0][:64], 'n1a': p['n1'][0][64:],
        'n1b': p['n1'][1][None, :],
        'n2': p['n2'][0], 'n2b': p['n2'][1][None, :],
    }
    return w


def kernel(nodes, edge_index, edge_features, params):
    f32 = jnp.float32
    coords = nodes[:, :3]
    vels = nodes[:, 3:6]
    feats = nodes[:, 6:]

    feats_pad = jnp.zeros((N_PAD, 128), f32).at[:N_NODES].set(feats)
    cexp = jnp.zeros((N_PAD, 16), f32).at[:N_NODES, :3].set(coords)
    vels_pad = jnp.zeros((N_PAD, 8), f32).at[:N_NODES, :3].set(vels)
    eft = jnp.zeros((8, E_PAD), f32).at[:4, :E_EDGES].set(edge_features.T)

    idx_s_flat = jnp.full((E_PAD,), DUMMY, jnp.int32).at[:E_EDGES].set(
        edge_index[0])
    idx_e_flat = jnp.full((E_PAD,), DUMMY, jnp.int32).at[:E_EDGES].set(
        edge_index[1])
    z128 = jnp.zeros((N_PAD, 128), f32)

    emb_w, emb_b = params['emb']
    table = _init_table(feats_pad, cexp, emb_w, emb_b[None, :])

    gather = _make_gather(CH_PER_W)
    scatter = _make_scatter(EPW // CHUNK_SC)
    idx_s = idx_s_flat.reshape(NW, CH_PER_W, CHUNK)
    idx_e = idx_e_flat.reshape(NW, CH_PER_W, CHUNK)
    idx_sc = idx_s_flat.reshape(NW, EPW // CHUNK_SC, CHUNK_SC)
    for p in params['blocks']:
        w = _prep_block_weights(p)
        xs, xe = gather(table, idx_s, idx_e)
        mc = _edge_mlp(xs, xe, eft, w)
        pa = scatter(mc, idx_sc, z128)
        table = _node_update(table, vels_pad, pa, w)

    return table[:N_NODES, 64:67]


# speedup vs baseline: 1.0962x; 1.0962x over previous
import functools

import jax
import jax.numpy as jnp
from jax import lax
from jax.experimental import pallas as pl
from jax.experimental.pallas import tpu as pltpu
from jax.experimental.pallas import tpu_sc as plsc

N_NODES = 10000
N_PAD = 10240
E_EDGES = 640000
NW = 32
CHUNK = 128
CH_PER_W = 158
EPW = CHUNK * CH_PER_W
E_PAD = EPW * NW
TW = 128
TE = 4096
TN = 1024
DUMMY = N_NODES

_MESH = dict(core_axis_name="c", subcore_axis_name="s", num_cores=2,
             num_subcores=16)


def _silu(x):
    return x * (1.0 / (1.0 + jnp.exp(-x)))


def _gather_body(ch, table, idx_s, idx_e, xs_out, xe_out,
                 idxs_v, idxe_v, bufs_a, bufs_b, sems_a, sems_b):
    cid = lax.axis_index("c")
    sid = lax.axis_index("s")
    wid = cid * 16 + sid
    pltpu.sync_copy(idx_s.at[wid], idxs_v)
    pltpu.sync_copy(idx_e.at[wid], idxe_v)
    base0 = wid * (ch * CHUNK)

    def fire(j, slot):
        pltpu.async_copy(table.at[idxs_v.at[j]], bufs_a[slot], sems_a[slot])
        pltpu.async_copy(table.at[idxe_v.at[j]], bufs_b[slot], sems_b[slot])

    def drain(j, slot):
        pltpu.make_async_copy(table.at[idxs_v.at[j]], bufs_a[slot],
                              sems_a[slot]).wait()
        pltpu.make_async_copy(table.at[idxe_v.at[j]], bufs_b[slot],
                              sems_b[slot]).wait()
        base = base0 + j * CHUNK
        pltpu.sync_copy(bufs_a[slot], xs_out.at[pl.ds(base, CHUNK)])
        pltpu.sync_copy(bufs_b[slot], xe_out.at[pl.ds(base, CHUNK)])

    fire(0, 0)
    fire(1, 1)

    def body(jj, carry):
        for b in range(2):
            j = jj * 2 + b
            drain(j, b)

            @pl.when(jj < ch // 2 - 1)
            def _():
                fire(j + 2, b)
        return carry

    lax.fori_loop(0, ch // 2, body, 0)


@functools.lru_cache(maxsize=None)
def _make_gather(ch):
    ne = NW * ch * CHUNK
    return functools.partial(
        pl.kernel,
        out_type=[jax.ShapeDtypeStruct((ne, TW), jnp.float32),
                  jax.ShapeDtypeStruct((ne, TW), jnp.float32)],
        mesh=plsc.VectorSubcoreMesh(**_MESH),
        scratch_types=[pltpu.VMEM((ch, CHUNK), jnp.int32),
                       pltpu.VMEM((ch, CHUNK), jnp.int32),
                       [pltpu.VMEM((CHUNK, TW), jnp.float32)] * 2,
                       [pltpu.VMEM((CHUNK, TW), jnp.float32)] * 2,
                       [pltpu.SemaphoreType.DMA] * 2,
                       [pltpu.SemaphoreType.DMA] * 2],
    )(functools.partial(_gather_body, ch))


CHUNK_SC = 64
CH_SC = EPW // CHUNK_SC


def _scatter_body(chsc, mc_in, idx_s, z128, p_out, acc, idx_bufs,
                  bufs, sems, isems):
    cid = lax.axis_index("c")
    sid = lax.axis_index("s")
    wid = cid * 16 + sid
    r0 = sid * (N_PAD // 16)
    nr = N_PAD // 16
    pltpu.sync_copy(z128.at[pl.ds(r0, nr)], acc.at[pl.ds(r0, nr)])
    plsc.subcore_barrier()
    base0 = wid * (chsc * CHUNK_SC)

    def fire(j, slot):
        base = base0 + j * CHUNK_SC
        pltpu.async_copy(mc_in.at[pl.ds(base, CHUNK_SC)], bufs[slot], sems[slot])
        pltpu.async_copy(idx_s.at[wid, j], idx_bufs[slot], isems[slot])

    def drain(j, slot):
        base = base0 + j * CHUNK_SC
        pltpu.make_async_copy(mc_in.at[pl.ds(base, CHUNK_SC)], bufs[slot],
                              sems[slot]).wait()
        pltpu.make_async_copy(idx_s.at[wid, j], idx_bufs[slot],
                              isems[slot]).wait()
        pltpu.sync_copy(bufs[slot], acc.at[idx_bufs[slot]], add=True)

    fire(0, 0)
    fire(1, 1)

    def body(jj, carry):
        for b in range(2):
            j = jj * 2 + b
            drain(j, b)

            @pl.when(jj < chsc // 2 - 1)
            def _():
                fire(j + 2, b)
        return carry

    lax.fori_loop(0, chsc // 2, body, 0)
    plsc.subcore_barrier()
    pltpu.sync_copy(acc.at[pl.ds(r0, nr)], p_out.at[cid, pl.ds(r0, nr)])


@functools.lru_cache(maxsize=None)
def _make_scatter(chsc):
    return functools.partial(
        pl.kernel,
        out_type=jax.ShapeDtypeStruct((2, N_PAD, 128), jnp.float32),
        mesh=plsc.VectorSubcoreMesh(**_MESH),
        scratch_types=[pltpu.VMEM_SHARED((N_PAD, 128), jnp.float32),
                       [pltpu.VMEM((CHUNK_SC,), jnp.int32)] * 2,
                       [pltpu.VMEM((CHUNK_SC, 128), jnp.float32)] * 2,
                       [pltpu.SemaphoreType.DMA] * 2,
                       [pltpu.SemaphoreType.DMA] * 2],
    )(functools.partial(_scatter_body, chsc))


def _init_body(feats, cexp, wemb, bemb, out):
    h0 = jnp.dot(feats[...], wemb[...],
                 preferred_element_type=jnp.float32) + bemb[...]
    out[...] = jnp.concatenate(
        [h0, cexp[:, :3], jnp.zeros((h0.shape[0], 61), jnp.float32)], axis=1)


def _init_table(feats_pad, cexp, wemb, bemb):
    return pl.pallas_call(
        _init_body,
        grid=(N_PAD // TN,),
        in_specs=[pl.BlockSpec((TN, 128), lambda i: (i, 0)),
                  pl.BlockSpec((TN, 16), lambda i: (i, 0)),
                  pl.BlockSpec((128, 64), lambda i: (0, 0)),
                  pl.BlockSpec((1, 64), lambda i: (0, 0))],
        out_specs=pl.BlockSpec((TN, TW), lambda i: (i, 0)),
        out_shape=jax.ShapeDtypeStruct((N_PAD, TW), jnp.float32),
    )(feats_pad, cexp, wemb, bemb)


def _edge_body(xs, xe, eft, w1h, w1e, w1n, w1f, b1, w2, b2, c1, c1b, c2p, c2bp,
               mc_out):
    hs = xs[:, :64]
    he = xe[:, :64]
    cd = xs[:, 64:67] - xe[:, 64:67]
    norm = jnp.sqrt(jnp.sum(cd * cd, axis=1, keepdims=True))
    mp = (jnp.dot(hs, w1h[...], preferred_element_type=jnp.float32)
          + jnp.dot(he, w1e[...], preferred_element_type=jnp.float32)
          + lax.dot_general(eft[...], w1f[...], (((0,), (0,)), ((), ())),
                            preferred_element_type=jnp.float32)
          + norm * w1n[...] + b1[...])
    m = _silu(mp)
    m = _silu(jnp.dot(m, w2[...], preferred_element_type=jnp.float32) + b2[...])
    ch = _silu(jnp.dot(m, c1[...], preferred_element_type=jnp.float32) + c1b[...])
    cm = jnp.dot(ch, c2p[...], preferred_element_type=jnp.float32) + c2bp[...]
    cdcm = cd * cm[:, :3]
    nrow = cd.shape[0]
    mc_out[...] = jnp.concatenate(
        [m, cdcm, jnp.ones((nrow, 1), jnp.float32),
         jnp.zeros((nrow, 60), jnp.float32)], axis=1)


def _edge_mlp(xs, xe, eft, w):
    ne = xs.shape[0]
    wspec = lambda shp: pl.BlockSpec(shp, lambda i: (0, 0))
    return pl.pallas_call(
        _edge_body,
        grid=(ne // TE,),
        in_specs=[pl.BlockSpec((TE, TW), lambda i: (i, 0)),
                  pl.BlockSpec((TE, TW), lambda i: (i, 0)),
                  pl.BlockSpec((8, TE), lambda i: (0, i)),
                  wspec((64, 64)), wspec((64, 64)), wspec((1, 64)),
                  wspec((8, 64)), wspec((1, 64)), wspec((64, 64)),
                  wspec((1, 64)), wspec((64, 64)), wspec((1, 64)),
                  wspec((64, 8)), wspec((1, 8))],
        out_specs=pl.BlockSpec((TE, 128), lambda i: (i, 0)),
        out_shape=jax.ShapeDtypeStruct((ne, 128), jnp.float32),
    )(xs, xe, eft, w['w1h'], w['w1e'], w['w1n'], w['w1f'], w['b1'],
      w['w2'], w['b2'], w['c1'], w['c1b'], w['c2p'], w['c2bp'])


def _node_body(tab, vels, pm0, pm1, v1, v1b, v2p, v2bp,
               n1h, n1a, n1b, n2, n2b, out):
    h = tab[:, :64]
    coords = tab[:, 64:67]
    agg = pm0[...] + pm1[...]
    aggm = agg[:, :64]
    aggc = agg[:, 64:67]
    cnt = jnp.maximum(agg[:, 67:68], 1.0)
    vs = _silu(jnp.dot(h, v1[...], preferred_element_type=jnp.float32) + v1b[...])
    vs = jnp.dot(vs, v2p[...], preferred_element_type=jnp.float32) + v2bp[...]
    newc = coords + aggc[:, :3] / cnt + vs[:, :1] * vels[:, :3]
    u = _silu(jnp.dot(h, n1h[...], preferred_element_type=jnp.float32)
              + jnp.dot(aggm, n1a[...], preferred_element_type=jnp.float32)
              + n1b[...])
    u = jnp.dot(u, n2[...], preferred_element_type=jnp.float32) + n2b[...]
    nrow = h.shape[0]
    out[...] = jnp.concatenate(
        [h + u, newc, jnp.zeros((nrow, 61), jnp.float32)], axis=1)


def _node_update(tab, vels_pad, pa, w):
    wspec = lambda shp: pl.BlockSpec(shp, lambda i: (0, 0))
    return pl.pallas_call(
        _node_body,
        grid=(N_PAD // TN,),
        in_specs=[pl.BlockSpec((TN, TW), lambda i: (i, 0)),
                  pl.BlockSpec((TN, 8), lambda i: (i, 0)),
                  pl.BlockSpec((TN, 128), lambda i: (i, 0)),
                  pl.BlockSpec((TN, 128), lambda i: (i, 0)),
                  wspec((64, 64)), wspec((1, 64)), wspec((64, 8)),
                  wspec((1, 8)), wspec((64, 64)), wspec((64, 64)),
                  wspec((1, 64)), wspec((64, 64)), wspec((1, 64))],
        out_specs=pl.BlockSpec((TN, TW), lambda i: (i, 0)),
        out_shape=jax.ShapeDtypeStruct((N_PAD, TW), jnp.float32),
    )(tab, vels_pad, pa[0], pa[1],
      w['v1'], w['v1b'], w['v2p'], w['v2bp'],
      w['n1h'], w['n1a'], w['n1b'], w['n2'], w['n2b'])


def _prep_block_weights(p):
    w1, b1 = p['e1']
    f32 = jnp.float32
    w = {
        'w1h': w1[:64],
        'w1e': w1[64:128],
        'w1n': w1[128:129],
        'w1f': jnp.zeros((8, 64), f32).at[:4].set(w1[129:133]),
        'b1': b1[None, :],
        'w2': p['e2'][0], 'b2': p['e2'][1][None, :],
        'c1': p['c1'][0], 'c1b': p['c1'][1][None, :],
        'c2p': jnp.zeros((64, 8), f32).at[:, :3].set(p['c2'][0]),
        'c2bp': jnp.zeros((1, 8), f32).at[0, :3].set(p['c2'][1]),
        'v1': p['v1'][0], 'v1b': p['v1'][1][None, :],
        'v2p': jnp.zeros((64, 8), f32).at[:, :1].set(p['v2'][0]),
        'v2bp': jnp.zeros((1, 8), f32).at[0, :1].set(p['v2'][1]),
        'n1h': p['n1'][0][:64], 'n1a': p['n1'][0][64:],
        'n1b': p['n1'][1][None, :],
        'n2': p['n2'][0], 'n2b': p['n2'][1][None, :],
    }
    return w


def kernel(nodes, edge_index, edge_features, params):
    f32 = jnp.float32
    coords = nodes[:, :3]
    vels = nodes[:, 3:6]
    feats = nodes[:, 6:]

    feats_pad = jnp.zeros((N_PAD, 128), f32).at[:N_NODES].set(feats)
    cexp = jnp.zeros((N_PAD, 16), f32).at[:N_NODES, :3].set(coords)
    vels_pad = jnp.zeros((N_PAD, 8), f32).at[:N_NODES, :3].set(vels)
    eft = jnp.zeros((8, E_PAD), f32).at[:4, :E_EDGES].set(edge_features.T)

    idx_s_flat = jnp.full((E_PAD,), DUMMY, jnp.int32).at[:E_EDGES].set(
        edge_index[0])
    idx_e_flat = jnp.full((E_PAD,), DUMMY, jnp.int32).at[:E_EDGES].set(
        edge_index[1])
    z128 = jnp.zeros((N_PAD, 128), f32)

    emb_w, emb_b = params['emb']
    table = _init_table(feats_pad, cexp, emb_w, emb_b[None, :])

    gather = _make_gather(CH_PER_W)
    scatter = _make_scatter(EPW // CHUNK_SC)
    idx_s = idx_s_flat.reshape(NW, CH_PER_W, CHUNK)
    idx_e = idx_e_flat.reshape(NW, CH_PER_W, CHUNK)
    idx_sc = idx_s_flat.reshape(NW, EPW // CHUNK_SC, CHUNK_SC)
    for p in params['blocks']:
        w = _prep_block_weights(p)
        xs, xe = gather(table, idx_s, idx_e)
        mc = _edge_mlp(xs, xe, eft, w)
        pa = scatter(mc, idx_sc, z128)
        table = _node_update(table, vels_pad, pa, w)

    return table[:N_NODES, 64:67]


# speedup vs baseline: 1.1953x; 1.0904x over previous
import functools

import jax
import jax.numpy as jnp
from jax import lax
from jax.experimental import pallas as pl
from jax.experimental.pallas import tpu as pltpu
from jax.experimental.pallas import tpu_sc as plsc

N_NODES = 10000
N_PAD = 10240
E_EDGES = 640000
NW = 32
CHUNK = 128
CH_PER_W = 158
EPW = CHUNK * CH_PER_W
E_PAD = EPW * NW
TW = 128
TE = 8192
TN = 1024
DUMMY = N_NODES

_MESH = dict(core_axis_name="c", subcore_axis_name="s", num_cores=2,
             num_subcores=16)


def _silu(x):
    return x * (1.0 / (1.0 + jnp.exp(-x)))


def _gather_body(ch, table, idx_s, idx_e, xs_out, xe_out,
                 idxs_v, idxe_v, bufs_a, bufs_b, sems_a, sems_b):
    cid = lax.axis_index("c")
    sid = lax.axis_index("s")
    wid = cid * 16 + sid
    pltpu.sync_copy(idx_s.at[wid], idxs_v)
    pltpu.sync_copy(idx_e.at[wid], idxe_v)
    base0 = wid * (ch * CHUNK)

    def fire(j, slot):
        pltpu.async_copy(table.at[idxs_v.at[j]], bufs_a[slot], sems_a[slot])
        pltpu.async_copy(table.at[idxe_v.at[j]], bufs_b[slot], sems_b[slot])

    def drain(j, slot):
        pltpu.make_async_copy(table.at[idxs_v.at[j]], bufs_a[slot],
                              sems_a[slot]).wait()
        pltpu.make_async_copy(table.at[idxe_v.at[j]], bufs_b[slot],
                              sems_b[slot]).wait()
        base = base0 + j * CHUNK
        pltpu.sync_copy(bufs_a[slot], xs_out.at[pl.ds(base, CHUNK)])
        pltpu.sync_copy(bufs_b[slot], xe_out.at[pl.ds(base, CHUNK)])

    fire(0, 0)
    fire(1, 1)

    def body(jj, carry):
        for b in range(2):
            j = jj * 2 + b
            drain(j, b)

            @pl.when(jj < ch // 2 - 1)
            def _():
                fire(j + 2, b)
        return carry

    lax.fori_loop(0, ch // 2, body, 0)


@functools.lru_cache(maxsize=None)
def _make_gather(ch):
    ne = NW * ch * CHUNK
    return functools.partial(
        pl.kernel,
        out_type=[jax.ShapeDtypeStruct((ne, TW), jnp.float32),
                  jax.ShapeDtypeStruct((ne, TW), jnp.float32)],
        mesh=plsc.VectorSubcoreMesh(**_MESH),
        scratch_types=[pltpu.VMEM((ch, CHUNK), jnp.int32),
                       pltpu.VMEM((ch, CHUNK), jnp.int32),
                       [pltpu.VMEM((CHUNK, TW), jnp.float32)] * 2,
                       [pltpu.VMEM((CHUNK, TW), jnp.float32)] * 2,
                       [pltpu.SemaphoreType.DMA] * 2,
                       [pltpu.SemaphoreType.DMA] * 2],
    )(functools.partial(_gather_body, ch))


CHUNK_SC = 64
CH_SC = EPW // CHUNK_SC


def _scatter_body(chsc, mc_in, idx_s, z128, p_out, acc, idx_bufs,
                  bufs, sems, isems):
    cid = lax.axis_index("c")
    sid = lax.axis_index("s")
    wid = cid * 16 + sid
    r0 = sid * (N_PAD // 16)
    nr = N_PAD // 16
    pltpu.sync_copy(z128.at[pl.ds(r0, nr)], acc.at[pl.ds(r0, nr)])
    plsc.subcore_barrier()
    base0 = wid * (chsc * CHUNK_SC)

    def fire(j, slot):
        base = base0 + j * CHUNK_SC
        pltpu.async_copy(mc_in.at[pl.ds(base, CHUNK_SC)], bufs[slot], sems[slot])
        pltpu.async_copy(idx_s.at[wid, j], idx_bufs[slot], isems[slot])

    def drain(j, slot):
        base = base0 + j * CHUNK_SC
        pltpu.make_async_copy(mc_in.at[pl.ds(base, CHUNK_SC)], bufs[slot],
                              sems[slot]).wait()
        pltpu.make_async_copy(idx_s.at[wid, j], idx_bufs[slot],
                              isems[slot]).wait()
        pltpu.sync_copy(bufs[slot], acc.at[idx_bufs[slot]], add=True)

    fire(0, 0)
    fire(1, 1)

    def body(jj, carry):
        for b in range(2):
            j = jj * 2 + b
            drain(j, b)

            @pl.when(jj < chsc // 2 - 1)
            def _():
                fire(j + 2, b)
        return carry

    lax.fori_loop(0, chsc // 2, body, 0)
    plsc.subcore_barrier()
    pltpu.sync_copy(acc.at[pl.ds(r0, nr)], p_out.at[cid, pl.ds(r0, nr)])


@functools.lru_cache(maxsize=None)
def _make_scatter(chsc):
    return functools.partial(
        pl.kernel,
        out_type=jax.ShapeDtypeStruct((2, N_PAD, 128), jnp.float32),
        mesh=plsc.VectorSubcoreMesh(**_MESH),
        scratch_types=[pltpu.VMEM_SHARED((N_PAD, 128), jnp.float32),
                       [pltpu.VMEM((CHUNK_SC,), jnp.int32)] * 2,
                       [pltpu.VMEM((CHUNK_SC, 128), jnp.float32)] * 2,
                       [pltpu.SemaphoreType.DMA] * 2,
                       [pltpu.SemaphoreType.DMA] * 2],
    )(functools.partial(_scatter_body, chsc))


def _init_body(feats, cexp, wemb, bemb, out):
    h0 = jnp.dot(feats[...], wemb[...],
                 preferred_element_type=jnp.float32) + bemb[...]
    out[...] = jnp.concatenate(
        [h0, cexp[:, :3], jnp.zeros((h0.shape[0], 61), jnp.float32)], axis=1)


def _init_table(feats_pad, cexp, wemb, bemb):
    return pl.pallas_call(
        _init_body,
        grid=(N_PAD // TN,),
        in_specs=[pl.BlockSpec((TN, 128), lambda i: (i, 0)),
                  pl.BlockSpec((TN, 16), lambda i: (i, 0)),
                  pl.BlockSpec((128, 64), lambda i: (0, 0)),
                  pl.BlockSpec((1, 64), lambda i: (0, 0))],
        out_specs=pl.BlockSpec((TN, TW), lambda i: (i, 0)),
        out_shape=jax.ShapeDtypeStruct((N_PAD, TW), jnp.float32),
    )(feats_pad, cexp, wemb, bemb)


def _edge_body(xs, xe, eft, w1h, w1e, w1n, w1f, b1, w2, b2, c1, c1b, c2p, c2bp,
               mc_out):
    hs = xs[:, :64]
    he = xe[:, :64]
    cd = xs[:, 64:67] - xe[:, 64:67]
    norm = jnp.sqrt(jnp.sum(cd * cd, axis=1, keepdims=True))
    mp = (jnp.dot(hs, w1h[...], preferred_element_type=jnp.float32)
          + jnp.dot(he, w1e[...], preferred_element_type=jnp.float32)
          + lax.dot_general(eft[...], w1f[...], (((0,), (0,)), ((), ())),
                            preferred_element_type=jnp.float32)
          + norm * w1n[...] + b1[...])
    m = _silu(mp)
    m = _silu(jnp.dot(m, w2[...], preferred_element_type=jnp.float32) + b2[...])
    ch = _silu(jnp.dot(m, c1[...], preferred_element_type=jnp.float32) + c1b[...])
    cm = jnp.dot(ch, c2p[...], preferred_element_type=jnp.float32) + c2bp[...]
    cdcm = cd * cm[:, :3]
    nrow = cd.shape[0]
    mc_out[...] = jnp.concatenate(
        [m, cdcm, jnp.ones((nrow, 1), jnp.float32),
         jnp.zeros((nrow, 60), jnp.float32)], axis=1)


def _edge_mlp(xs, xe, eft, w):
    ne = xs.shape[0]
    wspec = lambda shp: pl.BlockSpec(shp, lambda i: (0, 0))
    return pl.pallas_call(
        _edge_body,
        grid=(ne // TE,),
        in_specs=[pl.BlockSpec((TE, TW), lambda i: (i, 0)),
                  pl.BlockSpec((TE, TW), lambda i: (i, 0)),
                  pl.BlockSpec((8, TE), lambda i: (0, i)),
                  wspec((64, 64)), wspec((64, 64)), wspec((1, 64)),
                  wspec((8, 64)), wspec((1, 64)), wspec((64, 64)),
                  wspec((1, 64)), wspec((64, 64)), wspec((1, 64)),
                  wspec((64, 8)), wspec((1, 8))],
        out_specs=pl.BlockSpec((TE, 128), lambda i: (i, 0)),
        out_shape=jax.ShapeDtypeStruct((ne, 128), jnp.float32),
    )(xs, xe, eft, w['w1h'], w['w1e'], w['w1n'], w['w1f'], w['b1'],
      w['w2'], w['b2'], w['c1'], w['c1b'], w['c2p'], w['c2bp'])


def _node_body(tab, vels, pm0, pm1, v1, v1b, v2p, v2bp,
               n1h, n1a, n1b, n2, n2b, out):
    h = tab[:, :64]
    coords = tab[:, 64:67]
    agg = pm0[...] + pm1[...]
    aggm = agg[:, :64]
    aggc = agg[:, 64:67]
    cnt = jnp.maximum(agg[:, 67:68], 1.0)
    vs = _silu(jnp.dot(h, v1[...], preferred_element_type=jnp.float32) + v1b[...])
    vs = jnp.dot(vs, v2p[...], preferred_element_type=jnp.float32) + v2bp[...]
    newc = coords + aggc[:, :3] / cnt + vs[:, :1] * vels[:, :3]
    u = _silu(jnp.dot(h, n1h[...], preferred_element_type=jnp.float32)
              + jnp.dot(aggm, n1a[...], preferred_element_type=jnp.float32)
              + n1b[...])
    u = jnp.dot(u, n2[...], preferred_element_type=jnp.float32) + n2b[...]
    nrow = h.shape[0]
    out[...] = jnp.concatenate(
        [h + u, newc, jnp.zeros((nrow, 61), jnp.float32)], axis=1)


def _node_update(tab, vels_pad, pa, w):
    wspec = lambda shp: pl.BlockSpec(shp, lambda i: (0, 0))
    return pl.pallas_call(
        _node_body,
        grid=(N_PAD // TN,),
        in_specs=[pl.BlockSpec((TN, TW), lambda i: (i, 0)),
                  pl.BlockSpec((TN, 8), lambda i: (i, 0)),
                  pl.BlockSpec((TN, 128), lambda i: (i, 0)),
                  pl.BlockSpec((TN, 128), lambda i: (i, 0)),
                  wspec((64, 64)), wspec((1, 64)), wspec((64, 8)),
                  wspec((1, 8)), wspec((64, 64)), wspec((64, 64)),
                  wspec((1, 64)), wspec((64, 64)), wspec((1, 64))],
        out_specs=pl.BlockSpec((TN, TW), lambda i: (i, 0)),
        out_shape=jax.ShapeDtypeStruct((N_PAD, TW), jnp.float32),
    )(tab, vels_pad, pa[0], pa[1],
      w['v1'], w['v1b'], w['v2p'], w['v2bp'],
      w['n1h'], w['n1a'], w['n1b'], w['n2'], w['n2b'])


def _prep_block_weights(p):
    w1, b1 = p['e1']
    f32 = jnp.float32
    w = {
        'w1h': w1[:64],
        'w1e': w1[64:128],
        'w1n': w1[128:129],
        'w1f': jnp.zeros((8, 64), f32).at[:4].set(w1[129:133]),
        'b1': b1[None, :],
        'w2': p['e2'][0], 'b2': p['e2'][1][None, :],
        'c1': p['c1'][0], 'c1b': p['c1'][1][None, :],
        'c2p': jnp.zeros((64, 8), f32).at[:, :3].set(p['c2'][0]),
        'c2bp': jnp.zeros((1, 8), f32).at[0, :3].set(p['c2'][1]),
        'v1': p['v1'][0], 'v1b': p['v1'][1][None, :],
        'v2p': jnp.zeros((64, 8), f32).at[:, :1].set(p['v2'][0]),
        'v2bp': jnp.zeros((1, 8), f32).at[0, :1].set(p['v2'][1]),
        'n1h': p['n1'][0][:64], 'n1a': p['n1'][0][64:],
        'n1b': p['n1'][1][None, :],
        'n2': p['n2'][0], 'n2b': p['n2'][1][None, :],
    }
    return w


def kernel(nodes, edge_index, edge_features, params):
    f32 = jnp.float32
    coords = nodes[:, :3]
    vels = nodes[:, 3:6]
    feats = nodes[:, 6:]

    feats_pad = jnp.zeros((N_PAD, 128), f32).at[:N_NODES].set(feats)
    cexp = jnp.zeros((N_PAD, 16), f32).at[:N_NODES, :3].set(coords)
    vels_pad = jnp.zeros((N_PAD, 8), f32).at[:N_NODES, :3].set(vels)
    eft = jnp.zeros((8, E_PAD), f32).at[:4, :E_EDGES].set(edge_features.T)

    idx_s_flat = jnp.full((E_PAD,), DUMMY, jnp.int32).at[:E_EDGES].set(
        edge_index[0])
    idx_e_flat = jnp.full((E_PAD,), DUMMY, jnp.int32).at[:E_EDGES].set(
        edge_index[1])
    z128 = jnp.zeros((N_PAD, 128), f32)

    emb_w, emb_b = params['emb']
    table = _init_table(feats_pad, cexp, emb_w, emb_b[None, :])

    gather = _make_gather(CH_PER_W)
    scatter = _make_scatter(EPW // CHUNK_SC)
    idx_s = idx_s_flat.reshape(NW, CH_PER_W, CHUNK)
    idx_e = idx_e_flat.reshape(NW, CH_PER_W, CHUNK)
    idx_sc = idx_s_flat.reshape(NW, EPW // CHUNK_SC, CHUNK_SC)
    for p in params['blocks']:
        w = _prep_block_weights(p)
        xs, xe = gather(table, idx_s, idx_e)
        mc = _edge_mlp(xs, xe, eft, w)
        pa = scatter(mc, idx_sc, z128)
        table = _node_update(table, vels_pad, pa, w)

    return table[:N_NODES, 64:67]


# speedup vs baseline: 1.1961x; 1.0007x over previous
import functools

import jax
import jax.numpy as jnp
from jax import lax
from jax.experimental import pallas as pl
from jax.experimental.pallas import tpu as pltpu
from jax.experimental.pallas import tpu_sc as plsc

N_NODES = 10000
N_PAD = 10240
E_EDGES = 640000
NW = 32
CHUNK = 128
CH_PER_W = 158
EPW = CHUNK * CH_PER_W
E_PAD = EPW * NW
TW = 128
TE = 8192
TN = 2048
DUMMY = N_NODES

_MESH = dict(core_axis_name="c", subcore_axis_name="s", num_cores=2,
             num_subcores=16)


def _silu(x):
    return x * (1.0 / (1.0 + jnp.exp(-x)))


def _gather_body(ch, table, idx_s, idx_e, xs_out, xe_out,
                 idxs_v, idxe_v, bufs_a, bufs_b, sems_a, sems_b):
    cid = lax.axis_index("c")
    sid = lax.axis_index("s")
    wid = cid * 16 + sid
    pltpu.sync_copy(idx_s.at[wid], idxs_v)
    pltpu.sync_copy(idx_e.at[wid], idxe_v)
    base0 = wid * (ch * CHUNK)

    def fire(j, slot):
        pltpu.async_copy(table.at[idxs_v.at[j]], bufs_a[slot], sems_a[slot])
        pltpu.async_copy(table.at[idxe_v.at[j]], bufs_b[slot], sems_b[slot])

    def drain(j, slot):
        pltpu.make_async_copy(table.at[idxs_v.at[j]], bufs_a[slot],
                              sems_a[slot]).wait()
        pltpu.make_async_copy(table.at[idxe_v.at[j]], bufs_b[slot],
                              sems_b[slot]).wait()
        base = base0 + j * CHUNK
        pltpu.sync_copy(bufs_a[slot], xs_out.at[pl.ds(base, CHUNK)])
        pltpu.sync_copy(bufs_b[slot], xe_out.at[pl.ds(base, CHUNK)])

    fire(0, 0)
    fire(1, 1)

    def body(jj, carry):
        for b in range(2):
            j = jj * 2 + b
            drain(j, b)

            @pl.when(jj < ch // 2 - 1)
            def _():
                fire(j + 2, b)
        return carry

    lax.fori_loop(0, ch // 2, body, 0)


@functools.lru_cache(maxsize=None)
def _make_gather(ch):
    ne = NW * ch * CHUNK
    return functools.partial(
        pl.kernel,
        out_type=[jax.ShapeDtypeStruct((ne, TW), jnp.float32),
                  jax.ShapeDtypeStruct((ne, TW), jnp.float32)],
        mesh=plsc.VectorSubcoreMesh(**_MESH),
        scratch_types=[pltpu.VMEM((ch, CHUNK), jnp.int32),
                       pltpu.VMEM((ch, CHUNK), jnp.int32),
                       [pltpu.VMEM((CHUNK, TW), jnp.float32)] * 2,
                       [pltpu.VMEM((CHUNK, TW), jnp.float32)] * 2,
                       [pltpu.SemaphoreType.DMA] * 2,
                       [pltpu.SemaphoreType.DMA] * 2],
    )(functools.partial(_gather_body, ch))


CHUNK_SC = 64
CH_SC = EPW // CHUNK_SC


def _scatter_body(chsc, mc_in, idx_s, z128, p_out, acc, idx_bufs,
                  bufs, sems, isems):
    cid = lax.axis_index("c")
    sid = lax.axis_index("s")
    wid = cid * 16 + sid
    r0 = sid * (N_PAD // 16)
    nr = N_PAD // 16
    pltpu.sync_copy(z128.at[pl.ds(r0, nr)], acc.at[pl.ds(r0, nr)])
    plsc.subcore_barrier()
    base0 = wid * (chsc * CHUNK_SC)

    def fire(j, slot):
        base = base0 + j * CHUNK_SC
        pltpu.async_copy(mc_in.at[pl.ds(base, CHUNK_SC)], bufs[slot], sems[slot])
        pltpu.async_copy(idx_s.at[wid, j], idx_bufs[slot], isems[slot])

    def drain(j, slot):
        base = base0 + j * CHUNK_SC
        pltpu.make_async_copy(mc_in.at[pl.ds(base, CHUNK_SC)], bufs[slot],
                              sems[slot]).wait()
        pltpu.make_async_copy(idx_s.at[wid, j], idx_bufs[slot],
                              isems[slot]).wait()
        pltpu.sync_copy(bufs[slot], acc.at[idx_bufs[slot]], add=True)

    fire(0, 0)
    fire(1, 1)

    def body(jj, carry):
        for b in range(2):
            j = jj * 2 + b
            drain(j, b)

            @pl.when(jj < chsc // 2 - 1)
            def _():
                fire(j + 2, b)
        return carry

    lax.fori_loop(0, chsc // 2, body, 0)
    plsc.subcore_barrier()
    pltpu.sync_copy(acc.at[pl.ds(r0, nr)], p_out.at[cid, pl.ds(r0, nr)])


@functools.lru_cache(maxsize=None)
def _make_scatter(chsc):
    return functools.partial(
        pl.kernel,
        out_type=jax.ShapeDtypeStruct((2, N_PAD, 128), jnp.float32),
        mesh=plsc.VectorSubcoreMesh(**_MESH),
        scratch_types=[pltpu.VMEM_SHARED((N_PAD, 128), jnp.float32),
                       [pltpu.VMEM((CHUNK_SC,), jnp.int32)] * 2,
                       [pltpu.VMEM((CHUNK_SC, 128), jnp.float32)] * 2,
                       [pltpu.SemaphoreType.DMA] * 2,
                       [pltpu.SemaphoreType.DMA] * 2],
    )(functools.partial(_scatter_body, chsc))


def _init_body(feats, cexp, wemb, bemb, out):
    h0 = jnp.dot(feats[...], wemb[...],
                 preferred_element_type=jnp.float32) + bemb[...]
    out[...] = jnp.concatenate(
        [h0, cexp[:, :3], jnp.zeros((h0.shape[0], 61), jnp.float32)], axis=1)


def _init_table(feats_pad, cexp, wemb, bemb):
    return pl.pallas_call(
        _init_body,
        grid=(N_PAD // TN,),
        in_specs=[pl.BlockSpec((TN, 128), lambda i: (i, 0)),
                  pl.BlockSpec((TN, 16), lambda i: (i, 0)),
                  pl.BlockSpec((128, 64), lambda i: (0, 0)),
                  pl.BlockSpec((1, 64), lambda i: (0, 0))],
        out_specs=pl.BlockSpec((TN, TW), lambda i: (i, 0)),
        out_shape=jax.ShapeDtypeStruct((N_PAD, TW), jnp.float32),
    )(feats_pad, cexp, wemb, bemb)


def _edge_body(xs, xe, eft, w1h, w1e, w1n, w1f, b1, w2, b2, c1, c1b, c2p, c2bp,
               mc_out):
    hs = xs[:, :64]
    he = xe[:, :64]
    cd = xs[:, 64:67] - xe[:, 64:67]
    norm = jnp.sqrt(jnp.sum(cd * cd, axis=1, keepdims=True))
    mp = (jnp.dot(hs, w1h[...], preferred_element_type=jnp.float32)
          + jnp.dot(he, w1e[...], preferred_element_type=jnp.float32)
          + lax.dot_general(eft[...], w1f[...], (((0,), (0,)), ((), ())),
                            preferred_element_type=jnp.float32)
          + norm * w1n[...] + b1[...])
    m = _silu(mp)
    m = _silu(jnp.dot(m, w2[...], preferred_element_type=jnp.float32) + b2[...])
    ch = _silu(jnp.dot(m, c1[...], preferred_element_type=jnp.float32) + c1b[...])
    cm = jnp.dot(ch, c2p[...], preferred_element_type=jnp.float32) + c2bp[...]
    cdcm = cd * cm[:, :3]
    nrow = cd.shape[0]
    mc_out[...] = jnp.concatenate(
        [m, cdcm, jnp.ones((nrow, 1), jnp.float32),
         jnp.zeros((nrow, 60), jnp.float32)], axis=1)


def _edge_mlp(xs, xe, eft, w):
    ne = xs.shape[0]
    wspec = lambda shp: pl.BlockSpec(shp, lambda i: (0, 0))
    return pl.pallas_call(
        _edge_body,
        grid=(ne // TE,),
        in_specs=[pl.BlockSpec((TE, TW), lambda i: (i, 0)),
                  pl.BlockSpec((TE, TW), lambda i: (i, 0)),
                  pl.BlockSpec((8, TE), lambda i: (0, i)),
                  wspec((64, 64)), wspec((64, 64)), wspec((1, 64)),
                  wspec((8, 64)), wspec((1, 64)), wspec((64, 64)),
                  wspec((1, 64)), wspec((64, 64)), wspec((1, 64)),
                  wspec((64, 8)), wspec((1, 8))],
        out_specs=pl.BlockSpec((TE, 128), lambda i: (i, 0)),
        out_shape=jax.ShapeDtypeStruct((ne, 128), jnp.float32),
    )(xs, xe, eft, w['w1h'], w['w1e'], w['w1n'], w['w1f'], w['b1'],
      w['w2'], w['b2'], w['c1'], w['c1b'], w['c2p'], w['c2bp'])


def _node_body(tab, vels, pm0, pm1, v1, v1b, v2p, v2bp,
               n1h, n1a, n1b, n2, n2b, out):
    h = tab[:, :64]
    coords = tab[:, 64:67]
    agg = pm0[...] + pm1[...]
    aggm = agg[:, :64]
    aggc = agg[:, 64:67]
    cnt = jnp.maximum(agg[:, 67:68], 1.0)
    vs = _silu(jnp.dot(h, v1[...], preferred_element_type=jnp.float32) + v1b[...])
    vs = jnp.dot(vs, v2p[...], preferred_element_type=jnp.float32) + v2bp[...]
    newc = coords + aggc[:, :3] / cnt + vs[:, :1] * vels[:, :3]
    u = _silu(jnp.dot(h, n1h[...], preferred_element_type=jnp.float32)
              + jnp.dot(aggm, n1a[...], preferred_element_type=jnp.float32)
              + n1b[...])
    u = jnp.dot(u, n2[...], preferred_element_type=jnp.float32) + n2b[...]
    nrow = h.shape[0]
    out[...] = jnp.concatenate(
        [h + u, newc, jnp.zeros((nrow, 61), jnp.float32)], axis=1)


def _node_update(tab, vels_pad, pa, w):
    wspec = lambda shp: pl.BlockSpec(shp, lambda i: (0, 0))
    return pl.pallas_call(
        _node_body,
        grid=(N_PAD // TN,),
        in_specs=[pl.BlockSpec((TN, TW), lambda i: (i, 0)),
                  pl.BlockSpec((TN, 8), lambda i: (i, 0)),
                  pl.BlockSpec((TN, 128), lambda i: (i, 0)),
                  pl.BlockSpec((TN, 128), lambda i: (i, 0)),
                  wspec((64, 64)), wspec((1, 64)), wspec((64, 8)),
                  wspec((1, 8)), wspec((64, 64)), wspec((64, 64)),
                  wspec((1, 64)), wspec((64, 64)), wspec((1, 64))],
        out_specs=pl.BlockSpec((TN, TW), lambda i: (i, 0)),
        out_shape=jax.ShapeDtypeStruct((N_PAD, TW), jnp.float32),
    )(tab, vels_pad, pa[0], pa[1],
      w['v1'], w['v1b'], w['v2p'], w['v2bp'],
      w['n1h'], w['n1a'], w['n1b'], w['n2'], w['n2b'])


def _prep_block_weights(p):
    w1, b1 = p['e1']
    f32 = jnp.float32
    w = {
        'w1h': w1[:64],
        'w1e': w1[64:128],
        'w1n': w1[128:129],
        'w1f': jnp.zeros((8, 64), f32).at[:4].set(w1[129:133]),
        'b1': b1[None, :],
        'w2': p['e2'][0], 'b2': p['e2'][1][None, :],
        'c1': p['c1'][0], 'c1b': p['c1'][1][None, :],
        'c2p': jnp.zeros((64, 8), f32).at[:, :3].set(p['c2'][0]),
        'c2bp': jnp.zeros((1, 8), f32).at[0, :3].set(p['c2'][1]),
        'v1': p['v1'][0], 'v1b': p['v1'][1][None, :],
        'v2p': jnp.zeros((64, 8), f32).at[:, :1].set(p['v2'][0]),
        'v2bp': jnp.zeros((1, 8), f32).at[0, :1].set(p['v2'][1]),
        'n1h': p['n1'][0][:64], 'n1a': p['n1'][0][64:],
        'n1b': p['n1'][1][None, :],
        'n2': p['n2'][0], 'n2b': p['n2'][1][None, :],
    }
    return w


def kernel(nodes, edge_index, edge_features, params):
    f32 = jnp.float32
    coords = nodes[:, :3]
    vels = nodes[:, 3:6]
    feats = nodes[:, 6:]

    feats_pad = jnp.zeros((N_PAD, 128), f32).at[:N_NODES].set(feats)
    cexp = jnp.zeros((N_PAD, 16), f32).at[:N_NODES, :3].set(coords)
    vels_pad = jnp.zeros((N_PAD, 8), f32).at[:N_NODES, :3].set(vels)
    eft = jnp.zeros((8, E_PAD), f32).at[:4, :E_EDGES].set(edge_features.T)

    idx_s_flat = jnp.full((E_PAD,), DUMMY, jnp.int32).at[:E_EDGES].set(
        edge_index[0])
    idx_e_flat = jnp.full((E_PAD,), DUMMY, jnp.int32).at[:E_EDGES].set(
        edge_index[1])
    z128 = jnp.zeros((N_PAD, 128), f32)

    emb_w, emb_b = params['emb']
    table = _init_table(feats_pad, cexp, emb_w, emb_b[None, :])

    gather = _make_gather(CH_PER_W)
    scatter = _make_scatter(EPW // CHUNK_SC)
    idx_s = idx_s_flat.reshape(NW, CH_PER_W, CHUNK)
    idx_e = idx_e_flat.reshape(NW, CH_PER_W, CHUNK)
    idx_sc = idx_s_flat.reshape(NW, EPW // CHUNK_SC, CHUNK_SC)
    for p in params['blocks']:
        w = _prep_block_weights(p)
        xs, xe = gather(table, idx_s, idx_e)
        mc = _edge_mlp(xs, xe, eft, w)
        pa = scatter(mc, idx_sc, z128)
        table = _node_update(table, vels_pad, pa, w)

    return table[:N_NODES, 64:67]
